# Initial kernel scaffold; baseline (speedup 1.0000x reference)
import jax
import jax.numpy as jnp
from jax.experimental import pallas as pl


def kernel(tensor, sparsity, boost_percent, boost_tensor, inh_indices, inh_values):
    raise NotImplementedError("write your pallas kernel here")


# speedup vs baseline: 37.6685x; 37.6685x over previous
import functools

import jax
import jax.numpy as jnp
from jax import lax
from jax.experimental import pallas as pl
from jax.experimental.pallas import tpu as pltpu
from jax.experimental.pallas import tpu_sc as plsc

_B, _E = 32, 8192
_NNZ = 1024
_BISECT_ITERS = 52


def _stage1(k_ref, bp_ref, t_ref, bt_ref, bn_ref, bd_ref, bth_ref):
    t = t_ref[...]
    bt = bt_ref[...]
    bp = bp_ref[0]
    kmin = k_ref[0]
    mx = jnp.max(t, axis=1, keepdims=True)
    mn = jnp.min(t, axis=1, keepdims=True)
    closeness = (mx - t) / (mx - mn + 1e-6)
    bn = bt + closeness * bp
    bn_ref[...] = bn
    bd_ref[...] = jnp.where(t > 0.0, t, 0.0) + bn

    lo = jnp.min(bn, axis=1, keepdims=True)
    hi = jnp.max(bn, axis=1, keepdims=True) + 1.0

    def body(_, c):
        l, h = c
        mid = 0.5 * (l + h)
        cnt = jnp.sum((bn >= mid).astype(jnp.int32), axis=1, keepdims=True)
        p = cnt >= kmin
        return jnp.where(p, mid, l), jnp.where(p, h, mid)

    lo, hi = lax.fori_loop(0, _BISECT_ITERS, body, (lo, hi))
    bth_ref[...] = jnp.broadcast_to(lo, bth_ref.shape)


_SEG = (_B * _E) // 16
_EPT = _NNZ // 16


def _sc_scatter_body(bd_hbm, aff_hbm, aee_hbm, val_hbm, out_hbm,
                     aff_v, aee_v, val_v, g_v, table_sh, sem):
    c = lax.axis_index("c")
    s = lax.axis_index("s")

    @pl.when(c == 0)
    def _():
        pltpu.sync_copy(bd_hbm.at[pl.ds(s * _SEG, _SEG)],
                        table_sh.at[pl.ds(s * _SEG, _SEG)])
        pltpu.sync_copy(aff_hbm.at[pl.ds(s * _EPT, _EPT)], aff_v)
        pltpu.sync_copy(aee_hbm.at[pl.ds(s * _EPT, _EPT)], aee_v)
        pltpu.sync_copy(val_hbm.at[pl.ds(s * _EPT, _EPT)], val_v)
        pltpu.async_copy(bd_hbm.at[aff_v], g_v, sem).wait()
        for j in range(_EPT // 16):
            sl = pl.ds(j * 16, 16)
            g_v[sl] = g_v[sl] * val_v[sl]
        plsc.subcore_barrier()
        pltpu.sync_copy(g_v, table_sh.at[aee_v], add=True)
        plsc.subcore_barrier()
        pltpu.sync_copy(table_sh.at[pl.ds(s * _SEG, _SEG)],
                        out_hbm.at[pl.ds(s * _SEG, _SEG)])


@functools.cache
def _get_sc_scatter():
    return pl.kernel(
        _sc_scatter_body,
        out_type=jax.ShapeDtypeStruct((_B * _E,), jnp.float32),
        mesh=plsc.VectorSubcoreMesh(core_axis_name="c", subcore_axis_name="s"),
        scratch_types=[
            pltpu.VMEM((_EPT,), jnp.int32),
            pltpu.VMEM((_EPT,), jnp.int32),
            pltpu.VMEM((_EPT,), jnp.float32),
            pltpu.VMEM((_EPT,), jnp.float32),
            pltpu.VMEM_SHARED((_B * _E,), jnp.float32),
            pltpu.SemaphoreType.DMA,
        ],
    )


def _stage2(k_ref, dc_ref, inh_ref, bn_ref, bth_ref, val_ref,
            pm_ref, bo_ref, idx_ref, nv_ref, tv_ref):
    kmin = k_ref[0]
    kmax = k_ref[1]
    x = inh_ref[...]
    bn = bn_ref[...]

    lo = jnp.min(x, axis=1, keepdims=True)
    hi = jnp.max(x, axis=1, keepdims=True) + 1.0

    def body(_, c):
        l, h = c
        mid = 0.5 * (l + h)
        cnt = jnp.sum((x >= mid).astype(jnp.int32), axis=1, keepdims=True)
        p = cnt >= kmax
        return jnp.where(p, mid, l), jnp.where(p, h, mid)

    lo, hi = lax.fori_loop(0, _BISECT_ITERS, body, (lo, hi))
    thresh = lo
    pre = ((x >= thresh) & (x > 0.0)).astype(jnp.float32)
    aa = jnp.sum(pre)
    bmask = (bn >= bth_ref[:, 0:1]).astype(jnp.float32)
    cond = aa < kmin.astype(jnp.float32)
    pm = jnp.where(cond, jnp.maximum(pre, bmask), pre)
    pm_ref[...] = pm
    bo_ref[...] = jnp.where(pm > 0.0, 0.0, bn)

    x0 = x[0:1, :]
    j32 = lax.broadcasted_iota(jnp.int32, (32, 1), 0)
    start = jnp.clip(kmax - 20, 0, _E - 20)
    ranks = jnp.where(j32 == 0, 1, start + j32)

    mn0 = jnp.min(x0)
    mx0 = jnp.max(x0)
    lo2 = jnp.full((32, 1), mn0)
    hi2 = jnp.full((32, 1), mx0) + 1.0

    def body2(_, c):
        l, h = c
        mid = 0.5 * (l + h)
        cnt = jnp.sum((x0 >= mid).astype(jnp.int32), axis=1, keepdims=True)
        p = cnt >= ranks
        return jnp.where(p, mid, l), jnp.where(p, h, mid)

    lo2, hi2 = lax.fori_loop(0, _BISECT_ITERS, body2, (lo2, hi2))
    cnt_gt = jnp.sum((x0 >= hi2).astype(jnp.int32), axis=1, keepdims=True)
    jj = ranks - cnt_gt
    eq = (x0 >= lo2) & (x0 < hi2)
    lanes = lax.broadcasted_iota(jnp.int32, (1, _E), 1)
    loI = jnp.full((32, 1), -1, jnp.int32)
    hiI = jnp.full((32, 1), _E - 1, jnp.int32)

    def body3(_, c):
        l, h = c
        mid = l + (h - l) // 2
        cnt = jnp.sum((eq & (lanes <= mid)).astype(jnp.int32),
                      axis=1, keepdims=True)
        p = cnt >= jj
        return jnp.where(p, l, mid), jnp.where(p, mid, h)

    loI, hiI = lax.fori_loop(0, 13, body3, (loI, hiI))
    idx_ref[...] = jnp.broadcast_to(hiI, idx_ref.shape)

    v = val_ref[...]
    dc = dc_ref[0]
    maxabs = jnp.maximum(jnp.max(jnp.abs(v)), 0.01)
    sub = dc * maxabs
    nv_ref[...] = jnp.clip(v + sub, -1.0, 0.0)
    tv_ref[...] = jnp.broadcast_to(jnp.clip(-0.01 + sub, -1.0, 0.0),
                                   tv_ref.shape)


def kernel(tensor, sparsity, boost_percent, boost_tensor, inh_indices, inh_values):
    b, e = tensor.shape[0], tensor.shape[1]
    t2 = tensor.reshape(b, e)
    bt2 = boost_tensor.reshape(b, e)
    min_active = jnp.floor(sparsity[0] * e).astype(jnp.int32)
    max_active = jnp.ceil(sparsity[1] * e).astype(jnp.int32)
    kvals = jnp.stack([min_active, max_active])
    bp = boost_percent.reshape(1)

    smem = pl.BlockSpec(memory_space=pltpu.SMEM)
    vmem = pl.BlockSpec(memory_space=pltpu.VMEM)

    bn, bd, bth = pl.pallas_call(
        _stage1,
        in_specs=[smem, smem, vmem, vmem],
        out_specs=[vmem, vmem, vmem],
        out_shape=[
            jax.ShapeDtypeStruct((b, e), jnp.float32),
            jax.ShapeDtypeStruct((b, e), jnp.float32),
            jax.ShapeDtypeStruct((b, 128), jnp.float32),
        ],
    )(kvals, bp, t2, bt2)

    aff = inh_indices[0] * e + inh_indices[1]
    aee = inh_indices[4] * e + inh_indices[5]
    inh_flat = _get_sc_scatter()(bd.reshape(-1), aff, aee, inh_values)
    inhibited = inh_flat.reshape(b, e)

    nnz = inh_values.shape[0]
    conn = nnz + 20
    desired = 1e15 * e
    decay_c = jnp.float32(conn ** 2 / (conn ** 2 + desired * 2)).reshape(1)

    pm, bo, idx_out, nv_main, tail = pl.pallas_call(
        _stage2,
        in_specs=[smem, smem, vmem, vmem, vmem, vmem],
        out_specs=[vmem, vmem, vmem, vmem, vmem],
        out_shape=[
            jax.ShapeDtypeStruct((b, e), jnp.float32),
            jax.ShapeDtypeStruct((b, e), jnp.float32),
            jax.ShapeDtypeStruct((32, 128), jnp.int32),
            jax.ShapeDtypeStruct((nnz // 128, 128), jnp.float32),
            jax.ShapeDtypeStruct((1, 128), jnp.float32),
        ],
    )(kvals, decay_c, inhibited, bn, bth, inh_values.reshape(nnz // 128, 128))

    idt = inh_indices.dtype
    top = idx_out[0, 0].astype(idt)
    others = idx_out[1:21, 0].astype(idt)
    zeros20 = jnp.zeros((20,), dtype=idt)
    top20 = jnp.full((20,), top, dtype=idt)
    new_idx_cols = jnp.stack([zeros20, top20, zeros20, zeros20,
                              zeros20, others, zeros20, zeros20], axis=0)
    new_idx_f = jnp.concatenate([inh_indices, new_idx_cols], axis=1)
    new_vals_f = jnp.concatenate(
        [nv_main.reshape(-1), jnp.full((20,), tail[0, 0], dtype=inh_values.dtype)],
        axis=0)

    post_mask = pm.reshape(tensor.shape)
    boost_out = bo.reshape(tensor.shape)
    return (post_mask, boost_out, new_idx_f, new_vals_f)


# speedup vs baseline: 52.4269x; 1.3918x over previous
import functools

import jax
import jax.numpy as jnp
from jax import lax
from jax.experimental import pallas as pl
from jax.experimental.pallas import tpu as pltpu
from jax.experimental.pallas import tpu_sc as plsc

_B, _E = 32, 8192
_NNZ = 1024
_BISECT_ITERS = 40


def _hi_init(mx):
    return mx + jnp.abs(mx) * 1e-6 + 1e-30


def _kth_largest(x, k, iters):
    lo = jnp.min(x, axis=1, keepdims=True)
    hi = _hi_init(jnp.max(x, axis=1, keepdims=True))

    def body(_, c):
        l, h = c
        mid = 0.5 * (l + h)
        cnt = jnp.sum((x >= mid).astype(jnp.int32), axis=1, keepdims=True)
        p = cnt >= k
        return jnp.where(p, mid, l), jnp.where(p, h, mid)

    lo, hi = lax.fori_loop(0, iters, body, (lo, hi))
    return lo, hi


def _stage1(bp_ref, t_ref, bt_ref, bn_ref, bd_ref):
    t = t_ref[...]
    bt = bt_ref[...]
    bp = bp_ref[0]
    mx = jnp.max(t, axis=1, keepdims=True)
    mn = jnp.min(t, axis=1, keepdims=True)
    closeness = (mx - t) / (mx - mn + 1e-6)
    bn = bt + closeness * bp
    bn_ref[...] = bn
    bd_ref[...] = jnp.where(t > 0.0, t, 0.0) + bn


_SEG = (_B * _E) // 16
_EPT = _NNZ // 16


def _sc_scatter_body(bd_hbm, aff_hbm, aee_hbm, val_hbm, out_hbm,
                     aff_v, aee_v, val_v, g_v, table_sh, sem):
    c = lax.axis_index("c")
    s = lax.axis_index("s")

    @pl.when(c == 0)
    def _():
        pltpu.sync_copy(bd_hbm.at[pl.ds(s * _SEG, _SEG)],
                        table_sh.at[pl.ds(s * _SEG, _SEG)])
        pltpu.sync_copy(aff_hbm.at[pl.ds(s * _EPT, _EPT)], aff_v)
        pltpu.sync_copy(aee_hbm.at[pl.ds(s * _EPT, _EPT)], aee_v)
        pltpu.sync_copy(val_hbm.at[pl.ds(s * _EPT, _EPT)], val_v)
        pltpu.async_copy(bd_hbm.at[aff_v], g_v, sem).wait()
        for j in range(_EPT // 16):
            sl = pl.ds(j * 16, 16)
            g_v[sl] = g_v[sl] * val_v[sl]
        plsc.subcore_barrier()
        pltpu.sync_copy(g_v, table_sh.at[aee_v], add=True)
        plsc.subcore_barrier()
        pltpu.sync_copy(table_sh.at[pl.ds(s * _SEG, _SEG)],
                        out_hbm.at[pl.ds(s * _SEG, _SEG)])


@functools.cache
def _get_sc_scatter():
    return pl.kernel(
        _sc_scatter_body,
        out_type=jax.ShapeDtypeStruct((_B * _E,), jnp.float32),
        mesh=plsc.VectorSubcoreMesh(core_axis_name="c", subcore_axis_name="s"),
        scratch_types=[
            pltpu.VMEM((_EPT,), jnp.int32),
            pltpu.VMEM((_EPT,), jnp.int32),
            pltpu.VMEM((_EPT,), jnp.float32),
            pltpu.VMEM((_EPT,), jnp.float32),
            pltpu.VMEM_SHARED((_B * _E,), jnp.float32),
            pltpu.SemaphoreType.DMA,
        ],
    )


def _stage2(k_ref, dc_ref, inh_ref, bn_ref, val_ref,
            pm_ref, bo_ref, idx_ref, nv_ref, tv_ref):
    kmin = k_ref[0]
    kmax = k_ref[1]
    x = inh_ref[...]
    bn = bn_ref[...]

    thresh, _ = _kth_largest(x, kmax, _BISECT_ITERS)
    pre = ((x >= thresh) & (x > 0.0)).astype(jnp.float32)
    aa = jnp.sum(pre)
    pm_ref[...] = pre
    bo_ref[...] = jnp.where(pre > 0.0, 0.0, bn)

    cond = aa < kmin.astype(jnp.float32)

    @pl.when(cond)
    def _():
        bth, _ = _kth_largest(bn, kmin, _BISECT_ITERS)
        pm = jnp.maximum(pre, (bn >= bth).astype(jnp.float32))
        pm_ref[...] = pm
        bo_ref[...] = jnp.where(pm > 0.0, 0.0, bn)

    x0 = x[0:1, :]
    x0r = x0.reshape(64, 128)
    gidx = (lax.broadcasted_iota(jnp.int32, (64, 128), 0) * 128
            + lax.broadcasted_iota(jnp.int32, (64, 128), 1))

    mx0 = jnp.max(x0r)
    top = jnp.min(jnp.where(x0r == mx0, gidx, _E))
    idx_ref[0:1, :] = jnp.broadcast_to(top, (1, 128))

    t0 = thresh[0:1, 0:1]
    winners = x0r >= t0
    nwin = jnp.sum(winners.astype(jnp.int32))
    fast = (nwin == kmax) & (kmax >= 20)

    @pl.when(fast)
    def _():
        m = winners
        big = jnp.float32(3e38)
        for i in range(20):
            vals = jnp.where(m, x0r, big)
            mnv = jnp.min(vals)
            eqm = m & (x0r == mnv)
            pick = jnp.max(jnp.where(eqm, gidx, -1))
            idx_ref[20 - i:21 - i, :] = jnp.broadcast_to(pick, (1, 128))
            m = m & (gidx != pick)

    @pl.when(jnp.logical_not(fast))
    def _():
        j32 = lax.broadcasted_iota(jnp.int32, (32, 1), 0)
        start = jnp.clip(kmax - 20, 0, _E - 20)
        ranks = jnp.where(j32 == 0, 1, start + j32)
        lo2 = jnp.full((32, 1), jnp.min(x0))
        hi2 = _hi_init(jnp.full((32, 1), jnp.max(x0)))

        def body2(_, c):
            l, h = c
            mid = 0.5 * (l + h)
            cnt = jnp.sum((x0 >= mid).astype(jnp.int32), axis=1,
                          keepdims=True)
            p = cnt >= ranks
            return jnp.where(p, mid, l), jnp.where(p, h, mid)

        lo2, hi2 = lax.fori_loop(0, 52, body2, (lo2, hi2))
        cnt_gt = jnp.sum((x0 >= hi2).astype(jnp.int32), axis=1,
                         keepdims=True)
        jj = ranks - cnt_gt
        eq = (x0 >= lo2) & (x0 < hi2)
        lanes = lax.broadcasted_iota(jnp.int32, (1, _E), 1)
        loI = jnp.full((32, 1), -1, jnp.int32)
        hiI = jnp.full((32, 1), _E - 1, jnp.int32)

        def body3(_, c):
            l, h = c
            mid = l + (h - l) // 2
            cnt = jnp.sum((eq & (lanes <= mid)).astype(jnp.int32),
                          axis=1, keepdims=True)
            p = cnt >= jj
            return jnp.where(p, l, mid), jnp.where(p, mid, h)

        loI, hiI = lax.fori_loop(0, 13, body3, (loI, hiI))
        out = jnp.where(j32 == 0, top, hiI)
        idx_ref[...] = jnp.broadcast_to(out, idx_ref.shape)

    v = val_ref[...]
    dc = dc_ref[0]
    maxabs = jnp.maximum(jnp.max(jnp.abs(v)), 0.01)
    sub = dc * maxabs
    nv_ref[...] = jnp.clip(v + sub, -1.0, 0.0)
    tv_ref[...] = jnp.broadcast_to(jnp.clip(-0.01 + sub, -1.0, 0.0),
                                   tv_ref.shape)


def kernel(tensor, sparsity, boost_percent, boost_tensor, inh_indices, inh_values):
    b, e = tensor.shape[0], tensor.shape[1]
    t2 = tensor.reshape(b, e)
    bt2 = boost_tensor.reshape(b, e)
    min_active = jnp.floor(sparsity[0] * e).astype(jnp.int32)
    max_active = jnp.ceil(sparsity[1] * e).astype(jnp.int32)
    kvals = jnp.stack([min_active, max_active])
    bp = boost_percent.reshape(1)

    smem = pl.BlockSpec(memory_space=pltpu.SMEM)
    vmem = pl.BlockSpec(memory_space=pltpu.VMEM)

    bn, bd = pl.pallas_call(
        _stage1,
        in_specs=[smem, vmem, vmem],
        out_specs=[vmem, vmem],
        out_shape=[
            jax.ShapeDtypeStruct((b, e), jnp.float32),
            jax.ShapeDtypeStruct((b, e), jnp.float32),
        ],
    )(bp, t2, bt2)

    aff = inh_indices[0] * e + inh_indices[1]
    aee = inh_indices[4] * e + inh_indices[5]
    inh_flat = _get_sc_scatter()(bd.reshape(-1), aff, aee, inh_values)
    inhibited = inh_flat.reshape(b, e)

    nnz = inh_values.shape[0]
    conn = nnz + 20
    desired = 1e15 * e
    decay_c = jnp.float32(conn ** 2 / (conn ** 2 + desired * 2)).reshape(1)

    pm, bo, idx_out, nv_main, tail = pl.pallas_call(
        _stage2,
        in_specs=[smem, smem, vmem, vmem, vmem],
        out_specs=[vmem, vmem, vmem, vmem, vmem],
        out_shape=[
            jax.ShapeDtypeStruct((b, e), jnp.float32),
            jax.ShapeDtypeStruct((b, e), jnp.float32),
            jax.ShapeDtypeStruct((32, 128), jnp.int32),
            jax.ShapeDtypeStruct((nnz // 128, 128), jnp.float32),
            jax.ShapeDtypeStruct((1, 128), jnp.float32),
        ],
    )(kvals, decay_c, inhibited, bn, inh_values.reshape(nnz // 128, 128))

    idt = inh_indices.dtype
    top = idx_out[0, 0].astype(idt)
    others = idx_out[1:21, 0].astype(idt)
    zeros20 = jnp.zeros((20,), dtype=idt)
    top20 = jnp.full((20,), top, dtype=idt)
    new_idx_cols = jnp.stack([zeros20, top20, zeros20, zeros20,
                              zeros20, others, zeros20, zeros20], axis=0)
    new_idx_f = jnp.concatenate([inh_indices, new_idx_cols], axis=1)
    new_vals_f = jnp.concatenate(
        [nv_main.reshape(-1), jnp.full((20,), tail[0, 0], dtype=inh_values.dtype)],
        axis=0)

    post_mask = pm.reshape(tensor.shape)
    boost_out = bo.reshape(tensor.shape)
    return (post_mask, boost_out, new_idx_f, new_vals_f)


# speedup vs baseline: 56.5416x; 1.0785x over previous
import functools

import jax
import jax.numpy as jnp
from jax import lax
from jax.experimental import pallas as pl
from jax.experimental.pallas import tpu as pltpu
from jax.experimental.pallas import tpu_sc as plsc

_B, _E = 32, 8192
_NNZ = 1024
_BISECT_ITERS = 36


def _hi_init(mx):
    return mx + jnp.abs(mx) * 1e-6 + 1e-30


def _kth_largest(x, k, iters):
    lo = jnp.min(x, axis=1, keepdims=True)
    hi = _hi_init(jnp.max(x, axis=1, keepdims=True))

    def body(_, c):
        l, h = c
        mid = 0.5 * (l + h)
        cnt = jnp.sum((x >= mid).astype(jnp.int32), axis=1, keepdims=True)
        p = cnt >= k
        return jnp.where(p, mid, l), jnp.where(p, h, mid)

    lo, hi = lax.fori_loop(0, iters, body, (lo, hi))
    return lo, hi


def _stage1(bp_ref, t_ref, bt_ref, bn_ref, bd_ref):
    t = t_ref[...]
    bt = bt_ref[...]
    bp = bp_ref[0]
    mx = jnp.max(t, axis=1, keepdims=True)
    mn = jnp.min(t, axis=1, keepdims=True)
    closeness = (mx - t) / (mx - mn + 1e-6)
    bn = bt + closeness * bp
    bn_ref[...] = bn
    bd_ref[...] = jnp.where(t > 0.0, t, 0.0) + bn


_SEG = (_B * _E) // 16
_EPT = _NNZ // 16


def _sc_scatter_body(bd_hbm, idx_hbm, val_hbm, out_hbm,
                     r0_v, r1_v, aff_v, aee_v, val_v, g_v, table_sh, sem):
    c = lax.axis_index("c")
    s = lax.axis_index("s")

    @pl.when(c == 0)
    def _():
        pltpu.sync_copy(bd_hbm.at[pl.ds(s * _SEG, _SEG)],
                        table_sh.at[pl.ds(s * _SEG, _SEG)])
        sl0 = pl.ds(s * _EPT, _EPT)
        pltpu.sync_copy(idx_hbm.at[0, sl0], r0_v)
        pltpu.sync_copy(idx_hbm.at[1, sl0], r1_v)
        pltpu.sync_copy(idx_hbm.at[4, sl0], aff_v)
        pltpu.sync_copy(idx_hbm.at[5, sl0], aee_v)
        pltpu.sync_copy(val_hbm.at[sl0], val_v)
        for j in range(_EPT // 16):
            sl = pl.ds(j * 16, 16)
            aee_v[sl] = aff_v[sl] * _E + aee_v[sl]
            aff_v[sl] = r0_v[sl] * _E + r1_v[sl]
        pltpu.async_copy(bd_hbm.at[aff_v], g_v, sem).wait()
        for j in range(_EPT // 16):
            sl = pl.ds(j * 16, 16)
            g_v[sl] = g_v[sl] * val_v[sl]
        plsc.subcore_barrier()
        pltpu.sync_copy(g_v, table_sh.at[aee_v], add=True)
        plsc.subcore_barrier()
        pltpu.sync_copy(table_sh.at[pl.ds(s * _SEG, _SEG)],
                        out_hbm.at[pl.ds(s * _SEG, _SEG)])


@functools.cache
def _get_sc_scatter():
    return pl.kernel(
        _sc_scatter_body,
        out_type=jax.ShapeDtypeStruct((_B * _E,), jnp.float32),
        mesh=plsc.VectorSubcoreMesh(core_axis_name="c", subcore_axis_name="s"),
        scratch_types=[
            pltpu.VMEM((_EPT,), jnp.int32),
            pltpu.VMEM((_EPT,), jnp.int32),
            pltpu.VMEM((_EPT,), jnp.int32),
            pltpu.VMEM((_EPT,), jnp.int32),
            pltpu.VMEM((_EPT,), jnp.float32),
            pltpu.VMEM((_EPT,), jnp.float32),
            pltpu.VMEM_SHARED((_B * _E,), jnp.float32),
            pltpu.SemaphoreType.DMA,
        ],
    )


def _stage2(k_ref, dc_ref, inh_ref, bn_ref, val_ref,
            pm_ref, bo_ref, cols_ref, nv_ref):
    kmin = k_ref[0]
    kmax = k_ref[1]
    x = inh_ref[...]
    bn = bn_ref[...]

    thresh, _ = _kth_largest(x, kmax, _BISECT_ITERS)
    pre = ((x >= thresh) & (x > 0.0)).astype(jnp.float32)
    aa = jnp.sum(pre)
    pm_ref[...] = pre
    bo_ref[...] = jnp.where(pre > 0.0, 0.0, bn)

    cond = aa < kmin.astype(jnp.float32)

    @pl.when(cond)
    def _():
        bth, _ = _kth_largest(bn, kmin, _BISECT_ITERS)
        pm = jnp.maximum(pre, (bn >= bth).astype(jnp.float32))
        pm_ref[...] = pm
        bo_ref[...] = jnp.where(pm > 0.0, 0.0, bn)

    x0 = x[0:1, :]
    x0r = x0.reshape(64, 128)
    gidx = (lax.broadcasted_iota(jnp.int32, (64, 128), 0) * 128
            + lax.broadcasted_iota(jnp.int32, (64, 128), 1))

    mx0 = jnp.max(x0r)
    top = jnp.min(jnp.where(x0r == mx0, gidx, _E))

    lane = lax.broadcasted_iota(jnp.int32, (1, 128), 1)
    t0 = thresh[0:1, 0:1]
    winners = x0r >= t0
    nwin = jnp.sum(winners.astype(jnp.int32))
    fast = (nwin == kmax) & (kmax >= 20)

    def _write_cols(others_row):
        r8 = lax.broadcasted_iota(jnp.int32, (8, 128), 0)
        cols = jnp.where(r8 == 1, top,
                         jnp.where(r8 == 5,
                                   jnp.broadcast_to(others_row, (8, 128)),
                                   0))
        cols_ref[...] = cols

    @pl.when(fast)
    def _():
        m = winners
        big = jnp.float32(3e38)
        others_row = jnp.zeros((1, 128), jnp.int32)
        for i in range(20):
            vals = jnp.where(m, x0r, big)
            mnv = jnp.min(vals)
            eqm = m & (x0r == mnv)
            pick = jnp.max(jnp.where(eqm, gidx, -1))
            others_row = jnp.where(lane == 19 - i, pick, others_row)
            m = m & (gidx != pick)
        _write_cols(others_row)

    @pl.when(jnp.logical_not(fast))
    def _():
        j32 = lax.broadcasted_iota(jnp.int32, (32, 1), 0)
        start = jnp.clip(kmax - 20, 0, _E - 20)
        ranks = jnp.where(j32 == 0, 1, start + j32)
        lo2 = jnp.full((32, 1), jnp.min(x0))
        hi2 = _hi_init(jnp.full((32, 1), jnp.max(x0)))

        def body2(_, c):
            l, h = c
            mid = 0.5 * (l + h)
            cnt = jnp.sum((x0 >= mid).astype(jnp.int32), axis=1,
                          keepdims=True)
            p = cnt >= ranks
            return jnp.where(p, mid, l), jnp.where(p, h, mid)

        lo2, hi2 = lax.fori_loop(0, 52, body2, (lo2, hi2))
        cnt_gt = jnp.sum((x0 >= hi2).astype(jnp.int32), axis=1,
                         keepdims=True)
        jj = ranks - cnt_gt
        eq = (x0 >= lo2) & (x0 < hi2)
        lanes = lax.broadcasted_iota(jnp.int32, (1, _E), 1)
        loI = jnp.full((32, 1), -1, jnp.int32)
        hiI = jnp.full((32, 1), _E - 1, jnp.int32)

        def body3(_, c):
            l, h = c
            mid = l + (h - l) // 2
            cnt = jnp.sum((eq & (lanes <= mid)).astype(jnp.int32),
                          axis=1, keepdims=True)
            p = cnt >= jj
            return jnp.where(p, l, mid), jnp.where(p, mid, h)

        loI, hiI = lax.fori_loop(0, 13, body3, (loI, hiI))
        others_row = jnp.zeros((1, 128), jnp.int32)
        for j in range(20):
            others_row = jnp.where(lane == j, hiI[j + 1, 0], others_row)
        _write_cols(others_row)

    v = val_ref[...]
    dc = dc_ref[0]
    maxabs = jnp.maximum(jnp.max(jnp.abs(v)), 0.01)
    sub = dc * maxabs
    nv_ref[0:8, :] = jnp.clip(v + sub, -1.0, 0.0)
    nv_ref[8:9, :] = jnp.broadcast_to(jnp.clip(-0.01 + sub, -1.0, 0.0),
                                      (1, 128))


def kernel(tensor, sparsity, boost_percent, boost_tensor, inh_indices, inh_values):
    b, e = tensor.shape[0], tensor.shape[1]
    t2 = tensor.reshape(b, e)
    bt2 = boost_tensor.reshape(b, e)
    min_active = jnp.floor(sparsity[0] * e).astype(jnp.int32)
    max_active = jnp.ceil(sparsity[1] * e).astype(jnp.int32)
    kvals = jnp.stack([min_active, max_active])
    bp = boost_percent.reshape(1)

    smem = pl.BlockSpec(memory_space=pltpu.SMEM)
    vmem = pl.BlockSpec(memory_space=pltpu.VMEM)

    bn, bd = pl.pallas_call(
        _stage1,
        in_specs=[smem, vmem, vmem],
        out_specs=[vmem, vmem],
        out_shape=[
            jax.ShapeDtypeStruct((b, e), jnp.float32),
            jax.ShapeDtypeStruct((b, e), jnp.float32),
        ],
    )(bp, t2, bt2)

    inh_flat = _get_sc_scatter()(bd.reshape(-1), inh_indices, inh_values)
    inhibited = inh_flat.reshape(b, e)

    nnz = inh_values.shape[0]
    conn = nnz + 20
    desired = 1e15 * e
    decay_c = jnp.float32(conn ** 2 / (conn ** 2 + desired * 2)).reshape(1)

    pm, bo, cols, nv9 = pl.pallas_call(
        _stage2,
        in_specs=[smem, smem, vmem, vmem, vmem],
        out_specs=[vmem, vmem, vmem, vmem],
        out_shape=[
            jax.ShapeDtypeStruct((b, e), jnp.float32),
            jax.ShapeDtypeStruct((b, e), jnp.float32),
            jax.ShapeDtypeStruct((8, 128), jnp.int32),
            jax.ShapeDtypeStruct((9, 128), jnp.float32),
        ],
    )(kvals, decay_c, inhibited, bn, inh_values.reshape(nnz // 128, 128))

    new_idx_f = jnp.concatenate(
        [inh_indices, cols[:, :20].astype(inh_indices.dtype)], axis=1)
    new_vals_f = nv9.reshape(-1)[:nnz + 20].astype(inh_values.dtype)

    post_mask = pm.reshape(tensor.shape)
    boost_out = bo.reshape(tensor.shape)
    return (post_mask, boost_out, new_idx_f, new_vals_f)


# speedup vs baseline: 59.1918x; 1.0469x over previous
import functools

import jax
import jax.numpy as jnp
from jax import lax
from jax.experimental import pallas as pl
from jax.experimental.pallas import tpu as pltpu
from jax.experimental.pallas import tpu_sc as plsc

_B, _E = 32, 8192
_NNZ = 1024
_BISECT_ITERS = 32


def _hi_init(mx):
    return mx + jnp.abs(mx) * 1e-6 + 1e-30


def _kth_largest(x, k, iters):
    lo = jnp.min(x, axis=1, keepdims=True)
    hi = _hi_init(jnp.max(x, axis=1, keepdims=True))

    def body(_, c):
        l, h = c
        mid = 0.5 * (l + h)
        cnt = jnp.sum((x >= mid).astype(jnp.int32), axis=1, keepdims=True)
        p = cnt >= k
        return jnp.where(p, mid, l), jnp.where(p, h, mid)

    lo, hi = lax.fori_loop(0, iters, body, (lo, hi))
    return lo, hi


def _stage1(bp_ref, t_ref, bt_ref, bn_ref, bd_ref):
    t = t_ref[...]
    bt = bt_ref[...]
    bp = bp_ref[0]
    mx = jnp.max(t, axis=1, keepdims=True)
    mn = jnp.min(t, axis=1, keepdims=True)
    closeness = (mx - t) / (mx - mn + 1e-6)
    bn = bt + closeness * bp
    bn_ref[...] = bn
    bd_ref[...] = jnp.where(t > 0.0, t, 0.0) + bn


_SEG = (_B * _E) // 16
_EPT = _NNZ // 16


def _sc_scatter_body(bd_hbm, idx_hbm, val_hbm, out_hbm,
                     i01_v, i45_v, aff_v, aee_v, val_v, g_v, table_sh,
                     sem_stg, sem_g):
    c = lax.axis_index("c")
    s = lax.axis_index("s")

    @pl.when(c == 0)
    def _():
        stg = pltpu.async_copy(bd_hbm.at[pl.ds(s * _SEG, _SEG)],
                               table_sh.at[pl.ds(s * _SEG, _SEG)], sem_stg)
        sl0 = pl.ds(s * _EPT, _EPT)
        pltpu.sync_copy(idx_hbm.at[0, sl0], i01_v)
        pltpu.sync_copy(idx_hbm.at[1, sl0], i45_v)
        pltpu.sync_copy(idx_hbm.at[4, sl0], aff_v)
        pltpu.sync_copy(idx_hbm.at[5, sl0], aee_v)
        pltpu.sync_copy(val_hbm.at[sl0], val_v)
        for j in range(_EPT // 16):
            sl = pl.ds(j * 16, 16)
            aee_v[sl] = aff_v[sl] * _E + aee_v[sl]
            aff_v[sl] = i01_v[sl] * _E + i45_v[sl]
        pltpu.async_copy(bd_hbm.at[aff_v], g_v, sem_g).wait()
        for j in range(_EPT // 16):
            sl = pl.ds(j * 16, 16)
            g_v[sl] = g_v[sl] * val_v[sl]
        stg.wait()
        plsc.subcore_barrier()
        pltpu.sync_copy(g_v, table_sh.at[aee_v], add=True)
        plsc.subcore_barrier()
        pltpu.sync_copy(table_sh.at[pl.ds(s * _SEG, _SEG)],
                        out_hbm.at[pl.ds(s * _SEG, _SEG)])


@functools.cache
def _get_sc_scatter():
    return pl.kernel(
        _sc_scatter_body,
        out_type=jax.ShapeDtypeStruct((_B * _E,), jnp.float32),
        mesh=plsc.VectorSubcoreMesh(core_axis_name="c", subcore_axis_name="s"),
        scratch_types=[
            pltpu.VMEM((_EPT,), jnp.int32),
            pltpu.VMEM((_EPT,), jnp.int32),
            pltpu.VMEM((_EPT,), jnp.int32),
            pltpu.VMEM((_EPT,), jnp.int32),
            pltpu.VMEM((_EPT,), jnp.float32),
            pltpu.VMEM((_EPT,), jnp.float32),
            pltpu.VMEM_SHARED((_B * _E,), jnp.float32),
            pltpu.SemaphoreType.DMA,
            pltpu.SemaphoreType.DMA,
        ],
    )


def _stage2(k_ref, dc_ref, inh_ref, bn_ref, val_ref,
            pm_ref, bo_ref, cols_ref, nv_ref):
    kmin = k_ref[0]
    kmax = k_ref[1]
    x = inh_ref[...]
    bn = bn_ref[...]

    thresh, _ = _kth_largest(x, kmax, _BISECT_ITERS)
    pre = ((x >= thresh) & (x > 0.0)).astype(jnp.float32)
    aa = jnp.sum(pre)
    pm_ref[...] = pre
    bo_ref[...] = jnp.where(pre > 0.0, 0.0, bn)

    cond = aa < kmin.astype(jnp.float32)

    @pl.when(cond)
    def _():
        bth, _ = _kth_largest(bn, kmin, _BISECT_ITERS)
        pm = jnp.maximum(pre, (bn >= bth).astype(jnp.float32))
        pm_ref[...] = pm
        bo_ref[...] = jnp.where(pm > 0.0, 0.0, bn)

    x0 = x[0:1, :]
    x0r = x0.reshape(64, 128)
    gidx = (lax.broadcasted_iota(jnp.int32, (64, 128), 0) * 128
            + lax.broadcasted_iota(jnp.int32, (64, 128), 1))

    mx0 = jnp.max(x0r)
    top = jnp.min(jnp.where(x0r == mx0, gidx, _E))

    lane = lax.broadcasted_iota(jnp.int32, (1, 128), 1)
    t0 = thresh[0:1, 0:1]
    winners = x0r >= t0
    nwin = jnp.sum(winners.astype(jnp.int32))
    fast = (nwin == kmax) & (kmax >= 20)

    def _write_cols(others_row):
        r8 = lax.broadcasted_iota(jnp.int32, (8, 128), 0)
        cols = jnp.where(r8 == 1, top,
                         jnp.where(r8 == 5,
                                   jnp.broadcast_to(others_row, (8, 128)),
                                   0))
        cols_ref[...] = cols

    @pl.when(fast)
    def _():
        m = winners
        big = jnp.float32(3e38)
        others_row = jnp.zeros((1, 128), jnp.int32)
        for i in range(20):
            vals = jnp.where(m, x0r, big)
            mnv = jnp.min(vals)
            eqm = m & (x0r == mnv)
            pick = jnp.max(jnp.where(eqm, gidx, -1))
            others_row = jnp.where(lane == 19 - i, pick, others_row)
            m = m & (gidx != pick)
        _write_cols(others_row)

    @pl.when(jnp.logical_not(fast))
    def _():
        j32 = lax.broadcasted_iota(jnp.int32, (32, 1), 0)
        start = jnp.clip(kmax - 20, 0, _E - 20)
        ranks = jnp.where(j32 == 0, 1, start + j32)
        lo2 = jnp.full((32, 1), jnp.min(x0))
        hi2 = _hi_init(jnp.full((32, 1), jnp.max(x0)))

        def body2(_, c):
            l, h = c
            mid = 0.5 * (l + h)
            cnt = jnp.sum((x0 >= mid).astype(jnp.int32), axis=1,
                          keepdims=True)
            p = cnt >= ranks
            return jnp.where(p, mid, l), jnp.where(p, h, mid)

        lo2, hi2 = lax.fori_loop(0, 52, body2, (lo2, hi2))
        cnt_gt = jnp.sum((x0 >= hi2).astype(jnp.int32), axis=1,
                         keepdims=True)
        jj = ranks - cnt_gt
        eq = (x0 >= lo2) & (x0 < hi2)
        lanes = lax.broadcasted_iota(jnp.int32, (1, _E), 1)
        loI = jnp.full((32, 1), -1, jnp.int32)
        hiI = jnp.full((32, 1), _E - 1, jnp.int32)

        def body3(_, c):
            l, h = c
            mid = l + (h - l) // 2
            cnt = jnp.sum((eq & (lanes <= mid)).astype(jnp.int32),
                          axis=1, keepdims=True)
            p = cnt >= jj
            return jnp.where(p, l, mid), jnp.where(p, mid, h)

        loI, hiI = lax.fori_loop(0, 13, body3, (loI, hiI))
        others_row = jnp.zeros((1, 128), jnp.int32)
        for j in range(20):
            others_row = jnp.where(lane == j, hiI[j + 1, 0], others_row)
        _write_cols(others_row)

    v = val_ref[...]
    dc = dc_ref[0]
    maxabs = jnp.maximum(jnp.max(jnp.abs(v)), 0.01)
    sub = dc * maxabs
    nv_ref[0:8, :] = jnp.clip(v + sub, -1.0, 0.0)
    nv_ref[8:9, :] = jnp.broadcast_to(jnp.clip(-0.01 + sub, -1.0, 0.0),
                                      (1, 128))


def kernel(tensor, sparsity, boost_percent, boost_tensor, inh_indices, inh_values):
    b, e = tensor.shape[0], tensor.shape[1]
    t2 = tensor.reshape(b, e)
    bt2 = boost_tensor.reshape(b, e)
    min_active = jnp.floor(sparsity[0] * e).astype(jnp.int32)
    max_active = jnp.ceil(sparsity[1] * e).astype(jnp.int32)
    kvals = jnp.stack([min_active, max_active])
    bp = boost_percent.reshape(1)

    smem = pl.BlockSpec(memory_space=pltpu.SMEM)
    vmem = pl.BlockSpec(memory_space=pltpu.VMEM)

    bn, bd = pl.pallas_call(
        _stage1,
        in_specs=[smem, vmem, vmem],
        out_specs=[vmem, vmem],
        out_shape=[
            jax.ShapeDtypeStruct((b, e), jnp.float32),
            jax.ShapeDtypeStruct((b, e), jnp.float32),
        ],
    )(bp, t2, bt2)

    inh_flat = _get_sc_scatter()(bd.reshape(-1), inh_indices, inh_values)
    inhibited = inh_flat.reshape(b, e)

    nnz = inh_values.shape[0]
    conn = nnz + 20
    desired = 1e15 * e
    decay_c = jnp.float32(conn ** 2 / (conn ** 2 + desired * 2)).reshape(1)

    pm, bo, cols, nv9 = pl.pallas_call(
        _stage2,
        in_specs=[smem, smem, vmem, vmem, vmem],
        out_specs=[vmem, vmem, vmem, vmem],
        out_shape=[
            jax.ShapeDtypeStruct((b, e), jnp.float32),
            jax.ShapeDtypeStruct((b, e), jnp.float32),
            jax.ShapeDtypeStruct((8, 128), jnp.int32),
            jax.ShapeDtypeStruct((9, 128), jnp.float32),
        ],
    )(kvals, decay_c, inhibited, bn, inh_values.reshape(nnz // 128, 128))

    new_idx_f = jnp.concatenate(
        [inh_indices, cols[:, :20].astype(inh_indices.dtype)], axis=1)
    new_vals_f = nv9.reshape(-1)[:nnz + 20].astype(inh_values.dtype)

    post_mask = pm.reshape(tensor.shape)
    boost_out = bo.reshape(tensor.shape)
    return (post_mask, boost_out, new_idx_f, new_vals_f)
